# Initial kernel scaffold; baseline (speedup 1.0000x reference)
import jax
import jax.numpy as jnp
from jax.experimental import pallas as pl


def kernel(x, edge_index, batch, edge_attr, params):
    raise NotImplementedError("write your pallas kernel here")


# speedup vs baseline: 1.2441x; 1.2441x over previous
import functools

import jax
import jax.numpy as jnp
from jax import lax
from jax.experimental import pallas as pl
from jax.experimental.pallas import tpu as pltpu
from jax.experimental.pallas import tpu_sc as plsc

F32 = jnp.float32
HID = 16
CHUNK = 128
NW = 32


def _sc_gather(table, idx2):
    n_nodes, ncol = table.shape
    nrows = idx2.shape[0]
    base = nrows // NW
    rem = nrows - base * NW
    mesh = plsc.VectorSubcoreMesh(core_axis_name="c", subcore_axis_name="s")

    @functools.partial(
        pl.kernel,
        out_type=jax.ShapeDtypeStruct((nrows * CHUNK, ncol), F32),
        mesh=mesh,
        compiler_params=pltpu.CompilerParams(use_tc_tiling_on_sc=False),
        scratch_types=[
            pltpu.VMEM((CHUNK,), jnp.int32),
            pltpu.VMEM((CHUNK, ncol), F32),
            pltpu.SemaphoreType.DMA,
        ],
    )
    def gather_kernel(table_hbm, idx_hbm, out_hbm, idx_v, rows_v, sem):
        c = lax.axis_index("c")
        s = lax.axis_index("s")
        w = s * 2 + c

        def body(j, carry):
            r = w + NW * j
            pltpu.sync_copy(idx_hbm.at[r], idx_v)
            pltpu.async_copy(table_hbm.at[idx_v], rows_v, sem).wait()
            pltpu.sync_copy(rows_v, out_hbm.at[pl.ds(r * CHUNK, CHUNK)])
            return carry

        lax.fori_loop(0, base, body, 0, unroll=False)
        if rem:
            @pl.when(w < rem)
            def _():
                body(base, 0)

    return gather_kernel(table, idx2)


def _sc_scatter(msg, dst2, n_nodes, with_counts):
    nrows = dst2.shape[0]
    base = nrows // NW
    rem = nrows - base * NW
    slab = n_nodes // 16
    piece = 125
    mesh = plsc.VectorSubcoreMesh(core_axis_name="c", subcore_axis_name="s")

    out_type = [jax.ShapeDtypeStruct((2, n_nodes, HID), F32)]
    scratch = [
        pltpu.VMEM((CHUNK,), jnp.int32),
        pltpu.VMEM((CHUNK, HID), F32),
        pltpu.VMEM_SHARED((n_nodes, HID), F32),
    ]
    if with_counts:
        out_type.append(jax.ShapeDtypeStruct((2, n_nodes, HID), F32))
        scratch.append(pltpu.VMEM((CHUNK, HID), F32))
        scratch.append(pltpu.VMEM_SHARED((n_nodes, HID), F32))

    @functools.partial(pl.kernel, out_type=out_type, mesh=mesh,
                       compiler_params=pltpu.CompilerParams(
                           use_tc_tiling_on_sc=False),
                       scratch_types=scratch)
    def scatter_kernel(msg_hbm, dst_hbm, *refs):
        if with_counts:
            out_hbm, cnt_hbm, idx_v, row_v, acc, one_v, cacc = refs
        else:
            out_hbm, idx_v, row_v, acc = refs
        c = lax.axis_index("c")
        s = lax.axis_index("s")
        w = s * 2 + c

        def fill(i, carry):
            row_v[i] = jnp.zeros((HID,), F32)
            if with_counts:
                one_v[i] = jnp.full((HID,), 1.0, F32)
            return carry

        lax.fori_loop(0, CHUNK, fill, 0, unroll=False)
        for q in range(slab // piece):
            dst_slice = pl.ds(s * slab + q * piece, piece)
            pltpu.sync_copy(row_v.at[pl.ds(0, piece)], acc.at[dst_slice])
            if with_counts:
                pltpu.sync_copy(row_v.at[pl.ds(0, piece)], cacc.at[dst_slice])
        plsc.subcore_barrier()

        def body(j, carry):
            r = w + NW * j
            pltpu.sync_copy(dst_hbm.at[r], idx_v)
            pltpu.sync_copy(msg_hbm.at[pl.ds(r * CHUNK, CHUNK)], row_v)
            pltpu.sync_copy(row_v, acc.at[idx_v], add=True)
            if with_counts:
                pltpu.sync_copy(one_v, cacc.at[idx_v], add=True)
            return carry

        lax.fori_loop(0, base, body, 0, unroll=False)
        if rem:
            @pl.when(w < rem)
            def _():
                body(base, 0)
        plsc.subcore_barrier()
        pltpu.sync_copy(acc.at[pl.ds(s * slab, slab)],
                        out_hbm.at[c, pl.ds(s * slab, slab)])
        if with_counts:
            pltpu.sync_copy(cacc.at[pl.ds(s * slab, slab)],
                            cnt_hbm.at[c, pl.ds(s * slab, slab)])

    return scatter_kernel(msg, dst2)


def _tc_edge(hg, ea, ew1, eb1, w2m, b2m):
    n_edges, in_c = hg.shape
    tile = 2000
    grid = n_edges // tile

    def body(hg_ref, ea_ref, ew1_ref, eb1_ref, w2_ref, b2_ref, out_ref):
        he = jnp.maximum(
            jnp.dot(ea_ref[...], ew1_ref[...], preferred_element_type=F32)
            + eb1_ref[...], 0.0)
        hgv = hg_ref[...]
        t = jnp.dot(hgv, w2_ref[...], preferred_element_type=F32)
        acc = jnp.dot(hgv, b2_ref[...], preferred_element_type=F32)
        for k in range(HID):
            acc = acc + he[:, k:k + 1] * t[:, k * HID:(k + 1) * HID]
        out_ref[...] = acc

    return pl.pallas_call(
        body,
        grid=(grid,),
        in_specs=[
            pl.BlockSpec((tile, in_c), lambda i: (i, 0)),
            pl.BlockSpec((tile, 4), lambda i: (i, 0)),
            pl.BlockSpec((4, HID), lambda i: (0, 0)),
            pl.BlockSpec((1, HID), lambda i: (0, 0)),
            pl.BlockSpec((in_c, HID * HID), lambda i: (0, 0)),
            pl.BlockSpec((in_c, HID), lambda i: (0, 0)),
        ],
        out_specs=pl.BlockSpec((tile, HID), lambda i: (i, 0)),
        out_shape=jax.ShapeDtypeStruct((n_edges, HID), F32),
    )(hg, ea, ew1, eb1.reshape(1, HID), w2m, b2m)


def _tc_node(parts, cnts, h, root, bias, bn_g, bn_b, residual):
    n_nodes = h.shape[0]

    def body(p_ref, c_ref, h_ref, root_ref, bias_ref, g_ref, b_ref, out_ref):
        sums = p_ref[0] + p_ref[1]
        cnt = jnp.maximum(c_ref[0] + c_ref[1], 1.0)
        hv = h_ref[...]
        xn = (sums / cnt
              + jnp.dot(hv, root_ref[...], preferred_element_type=F32)
              + bias_ref[...])
        mean = jnp.mean(xn, axis=0, keepdims=True)
        ctr = xn - mean
        var = jnp.mean(ctr * ctr, axis=0, keepdims=True)
        xn = ctr * lax.rsqrt(var + 1e-5) * g_ref[...] + b_ref[...]
        xn = jnp.maximum(xn, 0.0)
        if residual:
            xn = xn + hv
        out_ref[...] = xn

    return pl.pallas_call(
        body,
        out_shape=jax.ShapeDtypeStruct((n_nodes, HID), F32),
    )(parts, cnts, h, root, bias.reshape(1, HID), bn_g.reshape(1, HID),
      bn_b.reshape(1, HID))


def _tc_pool(h, batch2, l1w, l1b, l2w, l2b, n_graphs):
    n_nodes = h.shape[0]

    def body(h_ref, b_ref, w1_ref, b1_ref, w2_ref, b2_ref, out_ref):
        gid = lax.broadcasted_iota(jnp.int32, (n_graphs, n_nodes), 0)
        oh = (gid == b_ref[...]).astype(F32)
        psum = jnp.dot(oh, h_ref[...], preferred_element_type=F32)
        cnt = jnp.maximum(jnp.sum(oh, axis=1, keepdims=True), 1.0)
        pooled = psum / cnt
        h1 = jnp.maximum(
            jnp.dot(pooled, w1_ref[...], preferred_element_type=F32)
            + b1_ref[...], 0.0)
        out_ref[...] = (jnp.dot(h1, w2_ref[...], preferred_element_type=F32)
                        + b2_ref[...])

    return pl.pallas_call(
        body,
        out_shape=jax.ShapeDtypeStruct((n_graphs, 1), F32),
    )(h, batch2, l1w, l1b.reshape(1, -1), l2w, l2b.reshape(1, 1))


def kernel(x, edge_index, batch, edge_attr, params):
    n_nodes = x.shape[0]
    n_graphs = 64
    src2 = edge_index[0].reshape(-1, CHUNK)
    dst2 = edge_index[1].reshape(-1, CHUNK)
    batch2 = batch.reshape(1, n_nodes)

    h = x
    cnts = None
    for l, p in enumerate(params["convs"]):
        in_c = h.shape[1]
        w2m = (p["ew2"].reshape(HID, in_c, HID)
               .transpose(1, 0, 2).reshape(in_c, HID * HID))
        b2m = p["eb2"].reshape(in_c, HID)
        hg = _sc_gather(h, src2)
        msg = _tc_edge(hg, edge_attr, p["ew1"], p["eb1"], w2m, b2m)
        if l == 0:
            parts, cnts = _sc_scatter(msg, dst2, n_nodes, with_counts=True)
        else:
            (parts,) = _sc_scatter(msg, dst2, n_nodes, with_counts=False)
        h = _tc_node(parts, cnts, h, p["root"], p["bias"], p["bn_g"],
                     p["bn_b"], residual=(l > 0))
    return _tc_pool(h, batch2, params["lin1_w"], params["lin1_b"],
                    params["lin2_w"], params["lin2_b"], n_graphs)


# speedup vs baseline: 2.9126x; 2.3411x over previous
import functools

import jax
import jax.numpy as jnp
from jax import lax
from jax.experimental import pallas as pl
from jax.experimental.pallas import tpu as pltpu
from jax.experimental.pallas import tpu_sc as plsc

F32 = jnp.float32
HID = 16
CHUNK = 128
NW = 32


def _sc_gather(table, idx2):
    n_nodes, ncol = table.shape
    nrows = idx2.shape[0]
    base = nrows // NW
    rem = nrows - base * NW
    mesh = plsc.VectorSubcoreMesh(core_axis_name="c", subcore_axis_name="s")

    @functools.partial(
        pl.kernel,
        out_type=jax.ShapeDtypeStruct((nrows * CHUNK, ncol), F32),
        mesh=mesh,
        compiler_params=pltpu.CompilerParams(use_tc_tiling_on_sc=False),
        scratch_types=[
            pltpu.VMEM((CHUNK,), jnp.int32),
            pltpu.VMEM((CHUNK, ncol), F32),
            pltpu.SemaphoreType.DMA,
        ],
    )
    def gather_kernel(table_hbm, idx_hbm, out_hbm, idx_v, rows_v, sem):
        c = lax.axis_index("c")
        s = lax.axis_index("s")
        w = s * 2 + c

        def body(j, carry):
            r = w + NW * j
            pltpu.sync_copy(idx_hbm.at[r], idx_v)
            pltpu.async_copy(table_hbm.at[idx_v], rows_v, sem).wait()
            pltpu.sync_copy(rows_v, out_hbm.at[pl.ds(r * CHUNK, CHUNK)])
            return carry

        lax.fori_loop(0, base, body, 0, unroll=False)
        if rem:
            @pl.when(w < rem)
            def _():
                body(base, 0)

    return gather_kernel(table, idx2)


def _sc_scatter(msg, dst2, n_nodes, with_counts):
    nrows = dst2.shape[0]
    base = nrows // NW
    rem = nrows - base * NW
    slab = n_nodes // 16
    piece = 125
    mesh = plsc.VectorSubcoreMesh(core_axis_name="c", subcore_axis_name="s")

    out_type = [jax.ShapeDtypeStruct((2, n_nodes, HID), F32)]
    scratch = [
        pltpu.VMEM((CHUNK,), jnp.int32),
        pltpu.VMEM((CHUNK, HID), F32),
        pltpu.VMEM_SHARED((n_nodes, HID), F32),
    ]
    if with_counts:
        out_type.append(jax.ShapeDtypeStruct((2, n_nodes, HID), F32))
        scratch.append(pltpu.VMEM((CHUNK, HID), F32))
        scratch.append(pltpu.VMEM_SHARED((n_nodes, HID), F32))

    @functools.partial(pl.kernel, out_type=out_type, mesh=mesh,
                       compiler_params=pltpu.CompilerParams(
                           use_tc_tiling_on_sc=False),
                       scratch_types=scratch)
    def scatter_kernel(msg_hbm, dst_hbm, *refs):
        if with_counts:
            out_hbm, cnt_hbm, idx_v, row_v, acc, one_v, cacc = refs
        else:
            out_hbm, idx_v, row_v, acc = refs
        c = lax.axis_index("c")
        s = lax.axis_index("s")
        w = s * 2 + c

        def fill(i, carry):
            row_v[i] = jnp.zeros((HID,), F32)
            if with_counts:
                one_v[i] = jnp.full((HID,), 1.0, F32)
            return carry

        lax.fori_loop(0, CHUNK, fill, 0, unroll=False)
        for q in range(slab // piece):
            dst_slice = pl.ds(s * slab + q * piece, piece)
            pltpu.sync_copy(row_v.at[pl.ds(0, piece)], acc.at[dst_slice])
            if with_counts:
                pltpu.sync_copy(row_v.at[pl.ds(0, piece)], cacc.at[dst_slice])
        plsc.subcore_barrier()

        def body(j, carry):
            r = w + NW * j
            pltpu.sync_copy(dst_hbm.at[r], idx_v)
            pltpu.sync_copy(msg_hbm.at[pl.ds(r * CHUNK, CHUNK)], row_v)
            pltpu.sync_copy(row_v, acc.at[idx_v], add=True)
            if with_counts:
                pltpu.sync_copy(one_v, cacc.at[idx_v], add=True)
            return carry

        lax.fori_loop(0, base, body, 0, unroll=False)
        if rem:
            @pl.when(w < rem)
            def _():
                body(base, 0)
        plsc.subcore_barrier()
        pltpu.sync_copy(acc.at[pl.ds(s * slab, slab)],
                        out_hbm.at[c, pl.ds(s * slab, slab)])
        if with_counts:
            pltpu.sync_copy(cacc.at[pl.ds(s * slab, slab)],
                            cnt_hbm.at[c, pl.ds(s * slab, slab)])

    return scatter_kernel(msg, dst2)


def _tc_edge(hg, ea, ew1, eb1, w2m, b2m):
    n_edges, in_c = hg.shape
    tile = 2000
    grid = n_edges // tile
    kk = jnp.arange(HID * HID, dtype=jnp.int32)
    rep = (kk[None, :] // HID == jnp.arange(HID, dtype=jnp.int32)[:, None]
           ).astype(F32)
    ssum = (kk[:, None] % HID == jnp.arange(HID, dtype=jnp.int32)[None, :]
            ).astype(F32)

    def body(hg_ref, ea_ref, ew1_ref, eb1_ref, w2_ref, b2_ref, rep_ref,
             ssum_ref, out_ref):
        he = jnp.maximum(
            jnp.dot(ea_ref[...], ew1_ref[...], preferred_element_type=F32)
            + eb1_ref[...], 0.0)
        hgv = hg_ref[...]
        t = jnp.dot(hgv, w2_ref[...], preferred_element_type=F32)
        her = jnp.dot(he, rep_ref[...], preferred_element_type=F32)
        out_ref[...] = (
            jnp.dot(her * t, ssum_ref[...], preferred_element_type=F32)
            + jnp.dot(hgv, b2_ref[...], preferred_element_type=F32))

    return pl.pallas_call(
        body,
        grid=(grid,),
        in_specs=[
            pl.BlockSpec((tile, in_c), lambda i: (i, 0)),
            pl.BlockSpec((tile, 4), lambda i: (i, 0)),
            pl.BlockSpec((4, HID), lambda i: (0, 0)),
            pl.BlockSpec((1, HID), lambda i: (0, 0)),
            pl.BlockSpec((in_c, HID * HID), lambda i: (0, 0)),
            pl.BlockSpec((in_c, HID), lambda i: (0, 0)),
            pl.BlockSpec((HID, HID * HID), lambda i: (0, 0)),
            pl.BlockSpec((HID * HID, HID), lambda i: (0, 0)),
        ],
        out_specs=pl.BlockSpec((tile, HID), lambda i: (i, 0)),
        out_shape=jax.ShapeDtypeStruct((n_edges, HID), F32),
    )(hg, ea, ew1, eb1.reshape(1, HID), w2m, b2m, rep, ssum)


def _tc_node(parts, cnts, h, root, bias, bn_g, bn_b, residual):
    n_nodes = h.shape[0]

    def body(p_ref, c_ref, h_ref, root_ref, bias_ref, g_ref, b_ref, out_ref):
        sums = p_ref[0] + p_ref[1]
        cnt = jnp.maximum(c_ref[0] + c_ref[1], 1.0)
        hv = h_ref[...]
        xn = (sums / cnt
              + jnp.dot(hv, root_ref[...], preferred_element_type=F32)
              + bias_ref[...])
        mean = jnp.mean(xn, axis=0, keepdims=True)
        ctr = xn - mean
        var = jnp.mean(ctr * ctr, axis=0, keepdims=True)
        xn = ctr * lax.rsqrt(var + 1e-5) * g_ref[...] + b_ref[...]
        xn = jnp.maximum(xn, 0.0)
        if residual:
            xn = xn + hv
        out_ref[...] = xn

    return pl.pallas_call(
        body,
        out_shape=jax.ShapeDtypeStruct((n_nodes, HID), F32),
    )(parts, cnts, h, root, bias.reshape(1, HID), bn_g.reshape(1, HID),
      bn_b.reshape(1, HID))


def _tc_pool(h, batch2, l1w, l1b, l2w, l2b, n_graphs):
    n_nodes = h.shape[0]

    def body(h_ref, b_ref, w1_ref, b1_ref, w2_ref, b2_ref, out_ref):
        gid = lax.broadcasted_iota(jnp.int32, (n_graphs, n_nodes), 0)
        oh = (gid == b_ref[...]).astype(F32)
        psum = jnp.dot(oh, h_ref[...], preferred_element_type=F32)
        cnt = jnp.maximum(jnp.sum(oh, axis=1, keepdims=True), 1.0)
        pooled = psum / cnt
        h1 = jnp.maximum(
            jnp.dot(pooled, w1_ref[...], preferred_element_type=F32)
            + b1_ref[...], 0.0)
        out_ref[...] = (jnp.dot(h1, w2_ref[...], preferred_element_type=F32)
                        + b2_ref[...])

    return pl.pallas_call(
        body,
        out_shape=jax.ShapeDtypeStruct((n_graphs, 1), F32),
    )(h, batch2, l1w, l1b.reshape(1, -1), l2w, l2b.reshape(1, 1))


def kernel(x, edge_index, batch, edge_attr, params):
    n_nodes = x.shape[0]
    n_graphs = 64
    src2 = edge_index[0].reshape(-1, CHUNK)
    dst2 = edge_index[1].reshape(-1, CHUNK)
    batch2 = batch.reshape(1, n_nodes)

    h = x
    cnts = None
    for l, p in enumerate(params["convs"]):
        in_c = h.shape[1]
        w2m = (p["ew2"].reshape(HID, in_c, HID)
               .transpose(1, 0, 2).reshape(in_c, HID * HID))
        b2m = p["eb2"].reshape(in_c, HID)
        hg = _sc_gather(h, src2)
        msg = _tc_edge(hg, edge_attr, p["ew1"], p["eb1"], w2m, b2m)
        if l == 0:
            parts, cnts = _sc_scatter(msg, dst2, n_nodes, with_counts=True)
        else:
            (parts,) = _sc_scatter(msg, dst2, n_nodes, with_counts=False)
        h = _tc_node(parts, cnts, h, p["root"], p["bias"], p["bn_g"],
                     p["bn_b"], residual=(l > 0))
    return _tc_pool(h, batch2, params["lin1_w"], params["lin1_b"],
                    params["lin2_w"], params["lin2_b"], n_graphs)


# speedup vs baseline: 4.2535x; 1.4604x over previous
import functools

import jax
import jax.numpy as jnp
from jax import lax
from jax.experimental import pallas as pl
from jax.experimental.pallas import tpu as pltpu
from jax.experimental.pallas import tpu_sc as plsc

F32 = jnp.float32
HID = 16
CHUNK = 128
NW = 32


def _sc_gather(table, idx2):
    n_nodes, ncol = table.shape
    nrows = idx2.shape[0]
    base = nrows // NW
    rem = nrows - base * NW
    mesh = plsc.VectorSubcoreMesh(core_axis_name="c", subcore_axis_name="s")

    @functools.partial(
        pl.kernel,
        out_type=jax.ShapeDtypeStruct((nrows * CHUNK, ncol), F32),
        mesh=mesh,
        compiler_params=pltpu.CompilerParams(use_tc_tiling_on_sc=False),
        scratch_types=[
            pltpu.VMEM((CHUNK,), jnp.int32),
            pltpu.VMEM((CHUNK, ncol), F32),
            pltpu.SemaphoreType.DMA,
        ],
    )
    def gather_kernel(table_hbm, idx_hbm, out_hbm, idx_v, rows_v, sem):
        c = lax.axis_index("c")
        s = lax.axis_index("s")
        w = s * 2 + c

        def body(j, carry):
            r = w + NW * j
            pltpu.sync_copy(idx_hbm.at[r], idx_v)
            pltpu.async_copy(table_hbm.at[idx_v], rows_v, sem).wait()
            pltpu.sync_copy(rows_v, out_hbm.at[pl.ds(r * CHUNK, CHUNK)])
            return carry

        lax.fori_loop(0, base, body, 0, unroll=False)
        if rem:
            @pl.when(w < rem)
            def _():
                body(base, 0)

    return gather_kernel(table, idx2)


def _sc_scatter(msg, dst2, n_nodes, with_counts):
    nrows = dst2.shape[0]
    base = nrows // NW
    rem = nrows - base * NW
    slab = n_nodes // 16
    piece = 125
    mesh = plsc.VectorSubcoreMesh(core_axis_name="c", subcore_axis_name="s")

    out_type = [jax.ShapeDtypeStruct((2, n_nodes, HID), F32)]
    scratch = [
        pltpu.VMEM((CHUNK,), jnp.int32),
        pltpu.VMEM((CHUNK, HID), F32),
        pltpu.VMEM_SHARED((n_nodes, HID), F32),
    ]
    if with_counts:
        out_type.append(jax.ShapeDtypeStruct((2, n_nodes, HID), F32))
        scratch.append(pltpu.VMEM((CHUNK, HID), F32))
        scratch.append(pltpu.VMEM_SHARED((n_nodes, HID), F32))

    @functools.partial(pl.kernel, out_type=out_type, mesh=mesh,
                       compiler_params=pltpu.CompilerParams(
                           use_tc_tiling_on_sc=False),
                       scratch_types=scratch)
    def scatter_kernel(msg_hbm, dst_hbm, *refs):
        if with_counts:
            out_hbm, cnt_hbm, idx_v, row_v, acc, one_v, cacc = refs
        else:
            out_hbm, idx_v, row_v, acc = refs
        c = lax.axis_index("c")
        s = lax.axis_index("s")
        w = s * 2 + c

        def fill(i, carry):
            row_v[i] = jnp.zeros((HID,), F32)
            if with_counts:
                one_v[i] = jnp.full((HID,), 1.0, F32)
            return carry

        lax.fori_loop(0, CHUNK, fill, 0, unroll=False)
        for q in range(slab // piece):
            dst_slice = pl.ds(s * slab + q * piece, piece)
            pltpu.sync_copy(row_v.at[pl.ds(0, piece)], acc.at[dst_slice])
            if with_counts:
                pltpu.sync_copy(row_v.at[pl.ds(0, piece)], cacc.at[dst_slice])
        plsc.subcore_barrier()

        def body(j, carry):
            r = w + NW * j
            pltpu.sync_copy(dst_hbm.at[r], idx_v)
            pltpu.sync_copy(msg_hbm.at[pl.ds(r * CHUNK, CHUNK)], row_v)
            pltpu.sync_copy(row_v, acc.at[idx_v], add=True)
            if with_counts:
                pltpu.sync_copy(one_v, cacc.at[idx_v], add=True)
            return carry

        lax.fori_loop(0, base, body, 0, unroll=False)
        if rem:
            @pl.when(w < rem)
            def _():
                body(base, 0)
        plsc.subcore_barrier()
        pltpu.sync_copy(acc.at[pl.ds(s * slab, slab)],
                        out_hbm.at[c, pl.ds(s * slab, slab)])
        if with_counts:
            pltpu.sync_copy(cacc.at[pl.ds(s * slab, slab)],
                            cnt_hbm.at[c, pl.ds(s * slab, slab)])

    return scatter_kernel(msg, dst2)


BF16 = jnp.bfloat16


def _tc_edge(hg, ea, ew1, eb1, w2m, b2m, pack, tile_rows):
    n_edges, in_c = hg.shape
    rows = n_edges // pack
    grid = rows // tile_rows
    kk = jnp.arange(HID * HID, dtype=jnp.int32)
    ssum = (kk[:, None] % HID == jnp.arange(HID, dtype=jnp.int32)[None, :]
            ).astype(F32)
    eye = jnp.eye(pack, dtype=F32)
    ewrep = jnp.kron(eye, jnp.kron(ew1, jnp.ones((1, HID), F32))).astype(BF16)
    ebrep = jnp.tile(jnp.repeat(eb1, HID), pack).reshape(1, -1)
    w2big = jnp.kron(eye, w2m).astype(BF16)
    b2big = jnp.kron(eye, b2m).astype(BF16)
    sbig = jnp.kron(eye, ssum).astype(BF16)
    hg_p = hg.reshape(rows, pack * in_c)
    ea_p = ea.reshape(rows, pack * 4)
    wk = pack * HID * HID

    def body(hg_ref, ea_ref, ewr_ref, ebr_ref, w2_ref, b2_ref, s_ref,
             out_ref):
        hgv = hg_ref[...].astype(BF16)
        her = jnp.maximum(
            jnp.dot(ea_ref[...].astype(BF16), ewr_ref[...],
                    preferred_element_type=F32) + ebr_ref[...], 0.0)
        t = jnp.dot(hgv, w2_ref[...], preferred_element_type=F32)
        prod = her.astype(BF16) * t.astype(BF16)
        out_ref[...] = (
            jnp.dot(prod, s_ref[...], preferred_element_type=F32)
            + jnp.dot(hgv, b2_ref[...], preferred_element_type=F32))

    return pl.pallas_call(
        body,
        grid=(grid,),
        in_specs=[
            pl.BlockSpec((tile_rows, pack * in_c), lambda i: (i, 0)),
            pl.BlockSpec((tile_rows, pack * 4), lambda i: (i, 0)),
            pl.BlockSpec((pack * 4, wk), lambda i: (0, 0)),
            pl.BlockSpec((1, wk), lambda i: (0, 0)),
            pl.BlockSpec((pack * in_c, wk), lambda i: (0, 0)),
            pl.BlockSpec((pack * in_c, pack * HID), lambda i: (0, 0)),
            pl.BlockSpec((wk, pack * HID), lambda i: (0, 0)),
        ],
        out_specs=pl.BlockSpec((tile_rows, pack * HID), lambda i: (i, 0)),
        out_shape=jax.ShapeDtypeStruct((rows, pack * HID), F32),
    )(hg_p, ea_p, ewrep, ebrep, w2big, b2big, sbig).reshape(n_edges, HID)


def _tc_node(parts, cnts, h, root, bias, bn_g, bn_b, residual):
    n_nodes = h.shape[0]

    def body(p_ref, c_ref, h_ref, root_ref, bias_ref, g_ref, b_ref, out_ref):
        sums = p_ref[0] + p_ref[1]
        cnt = jnp.maximum(c_ref[0] + c_ref[1], 1.0)
        hv = h_ref[...]
        xn = (sums / cnt
              + jnp.dot(hv, root_ref[...], preferred_element_type=F32)
              + bias_ref[...])
        mean = jnp.mean(xn, axis=0, keepdims=True)
        ctr = xn - mean
        var = jnp.mean(ctr * ctr, axis=0, keepdims=True)
        xn = ctr * lax.rsqrt(var + 1e-5) * g_ref[...] + b_ref[...]
        xn = jnp.maximum(xn, 0.0)
        if residual:
            xn = xn + hv
        out_ref[...] = xn

    return pl.pallas_call(
        body,
        out_shape=jax.ShapeDtypeStruct((n_nodes, HID), F32),
    )(parts, cnts, h, root, bias.reshape(1, HID), bn_g.reshape(1, HID),
      bn_b.reshape(1, HID))


def _tc_pool(h, batch2, l1w, l1b, l2w, l2b, n_graphs):
    n_nodes = h.shape[0]

    def body(h_ref, b_ref, w1_ref, b1_ref, w2_ref, b2_ref, out_ref):
        gid = lax.broadcasted_iota(jnp.int32, (n_graphs, n_nodes), 0)
        oh = (gid == b_ref[...]).astype(F32)
        psum = jnp.dot(oh, h_ref[...], preferred_element_type=F32)
        cnt = jnp.maximum(jnp.sum(oh, axis=1, keepdims=True), 1.0)
        pooled = psum / cnt
        h1 = jnp.maximum(
            jnp.dot(pooled, w1_ref[...], preferred_element_type=F32)
            + b1_ref[...], 0.0)
        out_ref[...] = (jnp.dot(h1, w2_ref[...], preferred_element_type=F32)
                        + b2_ref[...])

    return pl.pallas_call(
        body,
        out_shape=jax.ShapeDtypeStruct((n_graphs, 1), F32),
    )(h, batch2, l1w, l1b.reshape(1, -1), l2w, l2b.reshape(1, 1))


def kernel(x, edge_index, batch, edge_attr, params):
    n_nodes = x.shape[0]
    n_graphs = 64
    src2 = edge_index[0].reshape(-1, CHUNK)
    dst2 = edge_index[1].reshape(-1, CHUNK)
    batch2 = batch.reshape(1, n_nodes)

    h = x
    cnts = None
    for l, p in enumerate(params["convs"]):
        in_c = h.shape[1]
        w2m = (p["ew2"].reshape(HID, in_c, HID)
               .transpose(1, 0, 2).reshape(in_c, HID * HID))
        b2m = p["eb2"].reshape(in_c, HID)
        hg = _sc_gather(h, src2)
        pack, tile_rows = (2, 2000) if in_c == 64 else (8, 1000)
        msg = _tc_edge(hg, edge_attr, p["ew1"], p["eb1"], w2m, b2m,
                       pack, tile_rows)
        if l == 0:
            parts, cnts = _sc_scatter(msg, dst2, n_nodes, with_counts=True)
        else:
            (parts,) = _sc_scatter(msg, dst2, n_nodes, with_counts=False)
        h = _tc_node(parts, cnts, h, p["root"], p["bias"], p["bn_g"],
                     p["bn_b"], residual=(l > 0))
    return _tc_pool(h, batch2, params["lin1_w"], params["lin1_b"],
                    params["lin2_w"], params["lin2_b"], n_graphs)
